# baseline (speedup 1.0000x reference)
import functools

import jax
import jax.numpy as jnp
from jax.experimental import pallas as pl
from jax.experimental.pallas import tpu as pltpu
from jax.experimental.pallas import tpu_sc as plsc

CAP = 100000
FDIM = 128
BATCH = 4096
ROWS = 10000
NBLK = CAP // ROWS
PAD = BATCH + 2 * ROWS

LW = 4000
NWORK = CAP // LW
_SC_LANES = 16


def _enqueue_kernel(scal_ref, vl_ref, q_ref, oq_ref, vs_ref):
    b = pl.program_id(0)
    s = b * ROWS
    p = scal_ref[0]

    @pl.when(b == 0)
    def _():
        vs_ref[pl.ds(ROWS, BATCH), :] = vl_ref[...]

    c0 = s - p
    c0 = jnp.where(c0 < 0, c0 + CAP, c0)
    has = (c0 < BATCH) | (c0 >= CAP - ROWS)

    @pl.when(has)
    def _():
        rows = jax.lax.broadcasted_iota(jnp.int32, (ROWS, 1), 0) + s
        m = rows - p
        m = jnp.where(m < 0, m + CAP, m)
        in_win = m < BATCH
        c = jnp.where(c0 >= CAP - ROWS, c0 - CAP, c0)
        o = jnp.clip(c + ROWS, 0, BATCH + ROWS)
        oq_ref[...] = jnp.where(in_win, vs_ref[pl.ds(o, ROWS), :], q_ref[...])

    @pl.when(jnp.logical_not(has))
    def _():
        oq_ref[...] = q_ref[...]


def _enqueue(experience_queue, vl_feat, scal):
    grid_spec = pltpu.PrefetchScalarGridSpec(
        num_scalar_prefetch=1,
        grid=(NBLK,),
        in_specs=[
            pl.BlockSpec((BATCH, FDIM), lambda b, sp: (0, 0)),
            pl.BlockSpec((ROWS, FDIM), lambda b, sp: (b, 0)),
        ],
        out_specs=pl.BlockSpec((ROWS, FDIM), lambda b, sp: (b, 0)),
        scratch_shapes=[pltpu.VMEM((PAD, FDIM), jnp.float32)],
    )
    return pl.pallas_call(
        _enqueue_kernel,
        grid_spec=grid_spec,
        compiler_params=pltpu.CompilerParams(
            dimension_semantics=("arbitrary",),
        ),
        out_shape=jax.ShapeDtypeStruct((CAP, FDIM), jnp.float32),
    )(scal, vl_feat, experience_queue)


def _label_sc_kernel(ql_hbm, lab_hbm, pv_hbm, out_hbm, buf_v, lab_v, p_v):
    wid = jax.lax.axis_index("s") * 2 + jax.lax.axis_index("c")

    @pl.when(wid < NWORK)
    def _():
        base = wid * LW
        pltpu.sync_copy(ql_hbm.at[pl.ds(base, LW)], buf_v)
        pltpu.sync_copy(lab_hbm, lab_v.at[pl.ds(_SC_LANES, BATCH)])
        pltpu.sync_copy(pv_hbm, p_v)
        p_s = p_v[...][0]
        lanes = jax.lax.iota(jnp.int32, _SC_LANES)

        def body(j, carry):
            m0 = base + j * _SC_LANES - p_s
            m0 = jnp.where(m0 < 0, m0 + CAP, m0)
            c = jnp.where(m0 >= CAP - _SC_LANES, m0 - CAP, m0)
            start = jnp.clip(c + _SC_LANES, 0, BATCH + _SC_LANES)
            vals = lab_v[pl.ds(start, _SC_LANES)]
            mlane = c + lanes
            win = (mlane >= 0) & (mlane < BATCH)
            cur = buf_v[pl.ds(j * _SC_LANES, _SC_LANES)]
            buf_v[pl.ds(j * _SC_LANES, _SC_LANES)] = jnp.where(win, vals, cur)
            return carry

        jax.lax.fori_loop(0, LW // _SC_LANES, body, 0)
        pltpu.sync_copy(buf_v, out_hbm.at[pl.ds(base, LW)])


_label_sc = functools.partial(
    pl.kernel,
    mesh=plsc.VectorSubcoreMesh(core_axis_name="c", subcore_axis_name="s"),
    out_type=jax.ShapeDtypeStruct((CAP,), jnp.float32),
    scratch_types=[
        pltpu.VMEM((LW,), jnp.float32),
        pltpu.VMEM((_SC_LANES + BATCH + _SC_LANES,), jnp.float32),
        pltpu.VMEM((_SC_LANES,), jnp.int32),
    ],
)(_label_sc_kernel)


def kernel(experience_queue, exp_label_queue, vl_feat, label, ptr):
    p = jnp.asarray(ptr, dtype=jnp.int32)
    scal = jnp.stack([p])
    new_queue = _enqueue(experience_queue, vl_feat, scal)
    p_vec = jnp.full((_SC_LANES,), p, dtype=jnp.int32)
    new_labels = _label_sc(
        exp_label_queue.reshape(CAP), label.reshape(BATCH), p_vec
    ).reshape(CAP, 1)
    new_ptr = (p + BATCH) % CAP
    is_full = jnp.where(new_ptr < p, 1, 0).astype(jnp.int64)
    is_empty = jnp.where(BATCH > 0, 0, 1).astype(jnp.int64)
    return new_queue, new_labels, jnp.asarray(new_ptr, dtype=jnp.int64), is_full, is_empty


# speedup vs baseline: 1.0032x; 1.0032x over previous
import functools

import jax
import jax.numpy as jnp
from jax.experimental import pallas as pl
from jax.experimental.pallas import tpu as pltpu
from jax.experimental.pallas import tpu_sc as plsc

CAP = 100000
FDIM = 128
BATCH = 4096
ROWS = 10000
NBLK = CAP // ROWS
PAD = BATCH + 2 * ROWS

LW = 4000
NWORK = CAP // LW
_SC_LANES = 16


def _enqueue_kernel(scal_ref, vl_ref, q_ref, oq_ref, vs_ref):
    b = pl.program_id(0)
    s = b * ROWS
    p = scal_ref[0]

    @pl.when(b == 0)
    def _():
        vs_ref[pl.ds(ROWS, BATCH), :] = vl_ref[...]

    c0 = s - p
    c0 = jnp.where(c0 < 0, c0 + CAP, c0)
    has = (c0 < BATCH) | (c0 >= CAP - ROWS)

    @pl.when(has)
    def _():
        rows = jax.lax.broadcasted_iota(jnp.int32, (ROWS, 1), 0) + s
        m = rows - p
        m = jnp.where(m < 0, m + CAP, m)
        in_win = m < BATCH
        c = jnp.where(c0 >= CAP - ROWS, c0 - CAP, c0)
        o = jnp.clip(c + ROWS, 0, BATCH + ROWS)
        oq_ref[...] = jnp.where(in_win, vs_ref[pl.ds(o, ROWS), :], q_ref[...])

    @pl.when(jnp.logical_not(has))
    def _():
        oq_ref[...] = q_ref[...]


def _enqueue(experience_queue, vl_feat, scal):
    grid_spec = pltpu.PrefetchScalarGridSpec(
        num_scalar_prefetch=1,
        grid=(NBLK,),
        in_specs=[
            pl.BlockSpec((BATCH, FDIM), lambda b, sp: (0, 0)),
            pl.BlockSpec((ROWS, FDIM), lambda b, sp: (b, 0)),
        ],
        out_specs=pl.BlockSpec((ROWS, FDIM), lambda b, sp: (b, 0)),
        scratch_shapes=[pltpu.VMEM((PAD, FDIM), jnp.float32)],
    )
    return pl.pallas_call(
        _enqueue_kernel,
        grid_spec=grid_spec,
        compiler_params=pltpu.CompilerParams(
            dimension_semantics=("arbitrary",),
        ),
        out_shape=jax.ShapeDtypeStruct((CAP, FDIM), jnp.float32),
    )(scal, vl_feat, experience_queue)


def _label_sc_kernel(ql_hbm, lab_hbm, pv_hbm, out_hbm, buf_v, lab_v, p_v):
    wid = jax.lax.axis_index("s") * 2 + jax.lax.axis_index("c")

    @pl.when(wid < NWORK)
    def _():
        base = wid * LW
        pltpu.sync_copy(ql_hbm.at[pl.ds(base, LW)], buf_v)
        pltpu.sync_copy(lab_hbm, lab_v.at[pl.ds(_SC_LANES, BATCH)])
        pltpu.sync_copy(pv_hbm, p_v)
        p_s = p_v[...][0]
        lanes = jax.lax.iota(jnp.int32, _SC_LANES)

        def body(j, carry):
            m0 = base + j * _SC_LANES - p_s
            m0 = jnp.where(m0 < 0, m0 + CAP, m0)
            c = jnp.where(m0 >= CAP - _SC_LANES, m0 - CAP, m0)
            start = jnp.clip(c + _SC_LANES, 0, BATCH + _SC_LANES)
            vals = lab_v[pl.ds(start, _SC_LANES)]
            mlane = c + lanes
            win = (mlane >= 0) & (mlane < BATCH)
            cur = buf_v[pl.ds(j * _SC_LANES, _SC_LANES)]
            buf_v[pl.ds(j * _SC_LANES, _SC_LANES)] = jnp.where(win, vals, cur)
            return carry

        jax.lax.fori_loop(0, LW // _SC_LANES, body, 0)
        pltpu.sync_copy(buf_v, out_hbm.at[pl.ds(base, LW)])


_label_sc = functools.partial(
    pl.kernel,
    mesh=plsc.VectorSubcoreMesh(core_axis_name="c", subcore_axis_name="s"),
    out_type=jax.ShapeDtypeStruct((CAP,), jnp.float32),
    scratch_types=[
        pltpu.VMEM((LW,), jnp.float32),
        pltpu.VMEM((_SC_LANES + BATCH + _SC_LANES,), jnp.float32),
        pltpu.VMEM((_SC_LANES,), jnp.int32),
    ],
)(_label_sc_kernel)


def kernel(experience_queue, exp_label_queue, vl_feat, label, ptr):
    p = jnp.asarray(ptr, dtype=jnp.int32)
    scal = jnp.stack([p])
    p_vec = jnp.full((_SC_LANES,), p, dtype=jnp.int32)
    new_labels = _label_sc(
        exp_label_queue.reshape(CAP), label.reshape(BATCH), p_vec
    ).reshape(CAP, 1)
    new_queue = _enqueue(experience_queue, vl_feat, scal)
    new_ptr = (p + BATCH) % CAP
    is_full = jnp.where(new_ptr < p, 1, 0).astype(jnp.int64)
    is_empty = jnp.where(BATCH > 0, 0, 1).astype(jnp.int64)
    return new_queue, new_labels, jnp.asarray(new_ptr, dtype=jnp.int64), is_full, is_empty
